# baseline (speedup 1.0000x reference)
import functools

import jax
import jax.numpy as jnp
from jax import lax
from jax.experimental import pallas as pl
from jax.experimental.pallas import tpu as pltpu, tpu_sc as plsc

VOCAB = 1000000
D = 64
NB, NS_SEQ = 4096, 200
B = NB * NS_SEQ
NC, NS = 2, 16
NW = NC * NS
NB_PER_W = NB // NW
B_PER_W = B // NW
CHUNK = 40
CPB = NS_SEQ // CHUNK
NCHUNK = B_PER_W // CHUNK
R = 3
L = 16

_mesh = plsc.VectorSubcoreMesh(
    core_axis_name="c", subcore_axis_name="s", num_cores=NC, num_subcores=NS
)


@functools.partial(
    pl.kernel,
    out_type=jax.ShapeDtypeStruct((NB, NS_SEQ, D), jnp.float32),
    mesh=_mesh,
    compiler_params=pltpu.CompilerParams(
        has_side_effects=pltpu.SideEffectType.PURE,
        skip_device_barrier=True,
    ),
    scratch_types=[
        pltpu.VMEM((B_PER_W + L,), jnp.int32),
        pltpu.VMEM((R, 48), jnp.int32),
        pltpu.VMEM((R, 48), jnp.int32),
        pltpu.VMEM((R, CHUNK, 2 * D), jnp.float32),
        pltpu.VMEM((R, CHUNK, D), jnp.float32),
        pltpu.SemaphoreType.DMA,
        pltpu.SemaphoreType.DMA,
        pltpu.SemaphoreType.DMA,
        pltpu.SemaphoreType.DMA,
        pltpu.SemaphoreType.DMA,
        pltpu.SemaphoreType.DMA,
    ],
)
def _emb_lookup(idx_hbm, table_hbm, out_hbm, xv, qbuf, obuf, pairs_v, rows_v,
                g0, g1, g2, s0, s1, s2):
    wid = lax.axis_index("s") * NC + lax.axis_index("c")
    base = wid * B_PER_W
    b_base = wid * NB_PER_W
    gsem = (g0, g1, g2)
    ssem = (s0, s1, s2)

    pltpu.sync_copy(idx_hbm.at[pl.ds(base, B_PER_W)], xv.at[pl.ds(0, B_PER_W)])

    def fire_gather(gi, r):
        for j in range((CHUNK + L - 1) // L):
            v = xv[pl.ds(gi * CHUNK + j * L, L)]
            obuf[r, pl.ds(j * L, L)] = (v & 1) << 6
            qbuf[r, pl.ds(j * L, L)] = v >> 1

        pltpu.make_async_copy(
            table_hbm.at[qbuf.at[r, pl.ds(0, CHUNK)]], pairs_v.at[r], gsem[r]
        ).start()

    def gather_wait(r):
        pltpu.make_async_copy(
            table_hbm.at[qbuf.at[r, pl.ds(0, CHUNK)]], pairs_v.at[r], gsem[r]
        ).wait()

    def store_desc(gi, r):
        bb = b_base + gi // CPB
        s0_ = (gi % CPB) * CHUNK
        out_sl = out_hbm.at[bb, pl.ds(s0_, CHUNK), :]
        return pltpu.make_async_copy(rows_v.at[r], out_sl, ssem[r])

    def extract(r):
        ovecs = [obuf[r, pl.ds(k * L, L)] for k in range((CHUNK + L - 1) // L)]
        for i in range(CHUNK):
            off = ovecs[i // L][i % L]
            for k in range(D // L):
                rows_v[r, i, pl.ds(k * L, L)] = pairs_v[r, i, pl.ds(off + k * L, L)]

    fire_gather(0, 0)

    @pl.loop(0, NCHUNK - 1, step=3)
    def _steady(i):
        for p in range(3):
            g = i + p
            r = p
            rn = (r + 1) % R

            @pl.when(g >= 2)
            def _drain():
                store_desc(g - 2, rn).wait()

            fire_gather(g + 1, rn)
            gather_wait(r)
            extract(r)
            store_desc(g, r).start()

    store_desc(NCHUNK - 3, 1).wait()
    gather_wait(0)
    extract(0)
    store_desc(NCHUNK - 1, 0).start()

    store_desc(NCHUNK - 2, 2).wait()
    store_desc(NCHUNK - 1, 0).wait()


def kernel(x, table):
    table2 = table.reshape(VOCAB // 2, 2 * D)
    return _emb_lookup(x.reshape(-1), table2)


# speedup vs baseline: 1.0182x; 1.0182x over previous
import functools

import jax
import jax.numpy as jnp
from jax import lax
from jax.experimental import pallas as pl
from jax.experimental.pallas import tpu as pltpu, tpu_sc as plsc

VOCAB = 1000000
D = 64
NB, NS_SEQ = 4096, 200
B = NB * NS_SEQ
NC, NS = 2, 16
NW = NC * NS
NB_PER_W = NB // NW
B_PER_W = B // NW
CHUNK = 40
CPB = NS_SEQ // CHUNK
NCHUNK = B_PER_W // CHUNK
R = 6
PF = 2
L = 16

_mesh = plsc.VectorSubcoreMesh(
    core_axis_name="c", subcore_axis_name="s", num_cores=NC, num_subcores=NS
)


@functools.partial(
    pl.kernel,
    out_type=jax.ShapeDtypeStruct((NB, NS_SEQ, D), jnp.float32),
    mesh=_mesh,
    compiler_params=pltpu.CompilerParams(
        has_side_effects=pltpu.SideEffectType.PURE,
        skip_device_barrier=True,
    ),
    scratch_types=[
        pltpu.VMEM((B_PER_W + L,), jnp.int32),
        pltpu.VMEM((R, 48), jnp.int32),
        pltpu.VMEM((R, 48), jnp.int32),
        pltpu.VMEM((R, CHUNK, 2 * D), jnp.float32),
        pltpu.VMEM((R, CHUNK, D), jnp.float32),
        pltpu.SemaphoreType.DMA,
        pltpu.SemaphoreType.DMA,
        pltpu.SemaphoreType.DMA,
        pltpu.SemaphoreType.DMA,
        pltpu.SemaphoreType.DMA,
        pltpu.SemaphoreType.DMA,
        pltpu.SemaphoreType.DMA,
        pltpu.SemaphoreType.DMA,
        pltpu.SemaphoreType.DMA,
        pltpu.SemaphoreType.DMA,
        pltpu.SemaphoreType.DMA,
        pltpu.SemaphoreType.DMA,
    ],
)
def _emb_lookup(idx_hbm, table_hbm, out_hbm, xv, qbuf, obuf, pairs_v, rows_v,
                g0, g1, g2, g3, g4, g5, s0, s1, s2, s3, s4, s5):
    wid = lax.axis_index("s") * NC + lax.axis_index("c")
    base = wid * B_PER_W
    b_base = wid * NB_PER_W
    gsem = (g0, g1, g2, g3, g4, g5)
    ssem = (s0, s1, s2, s3, s4, s5)

    pltpu.sync_copy(idx_hbm.at[pl.ds(base, B_PER_W)], xv.at[pl.ds(0, B_PER_W)])

    def fire_gather(gi, r):
        for j in range((CHUNK + L - 1) // L):
            v = xv[pl.ds(gi * CHUNK + j * L, L)]
            obuf[r, pl.ds(j * L, L)] = (v & 1) << 6
            qbuf[r, pl.ds(j * L, L)] = v >> 1

        pltpu.make_async_copy(
            table_hbm.at[qbuf.at[r, pl.ds(0, CHUNK)]], pairs_v.at[r], gsem[r]
        ).start()

    def gather_wait(r):
        pltpu.make_async_copy(
            table_hbm.at[qbuf.at[r, pl.ds(0, CHUNK)]], pairs_v.at[r], gsem[r]
        ).wait()

    def store_desc(gi, r):
        bb = b_base + gi // CPB
        s0_ = (gi % CPB) * CHUNK
        out_sl = out_hbm.at[bb, pl.ds(s0_, CHUNK), :]
        return pltpu.make_async_copy(rows_v.at[r], out_sl, ssem[r])

    def extract(r):
        ovecs = [obuf[r, pl.ds(k * L, L)] for k in range((CHUNK + L - 1) // L)]
        for i in range(CHUNK):
            off = ovecs[i // L][i % L]
            for k in range(D // L):
                rows_v[r, i, pl.ds(k * L, L)] = pairs_v[r, i, pl.ds(off + k * L, L)]

    for j in range(PF):
        fire_gather(j, j)

    @pl.loop(0, NCHUNK - 4, step=R)
    def _steady(i):
        for p in range(R):
            g = i + p
            r = p
            rn = (r + PF) % R

            @pl.when(g >= R - PF)
            def _drain():
                store_desc(g + PF - R, rn).wait()

            fire_gather(g + PF, rn)
            gather_wait(r)
            extract(r)
            store_desc(g, r).start()

    for g in range(NCHUNK - 4, NCHUNK):
        r = g % R
        rn = (r + PF) % R
        store_desc(g + PF - R, rn).wait()
        if g + PF < NCHUNK:
            fire_gather(g + PF, rn)
        gather_wait(r)
        extract(r)
        store_desc(g, r).start()

    for g in range(NCHUNK - (R - PF), NCHUNK):
        store_desc(g, g % R).wait()


def kernel(x, table):
    table2 = table.reshape(VOCAB // 2, 2 * D)
    return _emb_lookup(x.reshape(-1), table2)


# speedup vs baseline: 1.1428x; 1.1224x over previous
import functools

import jax
import jax.numpy as jnp
from jax import lax
from jax.experimental import pallas as pl
from jax.experimental.pallas import tpu as pltpu, tpu_sc as plsc

VOCAB = 1000000
D = 64
B = 4096 * 200
NC, NS = 2, 16
NW = NC * NS
B_PER_W = B // NW
CHUNK = 128
NCHUNK = B_PER_W // CHUNK
K = 4
G = NCHUNK // K
R = 3

_mesh = plsc.VectorSubcoreMesh(
    core_axis_name="c", subcore_axis_name="s", num_cores=NC, num_subcores=NS
)


@functools.partial(
    pl.kernel,
    out_type=jax.ShapeDtypeStruct((B, D), jnp.float32),
    mesh=_mesh,
    compiler_params=pltpu.CompilerParams(use_tc_tiling_on_sc=False),
)
def _emb_lookup(idx_hbm, table_hbm, out_hbm):
    wid = lax.axis_index("s") * NC + lax.axis_index("c")
    base = wid * B_PER_W

    def body(idx_v, rows_v, g0, g1, g2, s0, s1, s2):
        gsem = (g0, g1, g2)
        ssem = (s0, s1, s2)

        pltpu.sync_copy(idx_hbm.at[pl.ds(base, B_PER_W)], idx_v)

        def gather_desc(gi, r, b):
            idx_sl = idx_v.at[pl.ds((gi * K + b) * CHUNK, CHUNK)]
            return pltpu.make_async_copy(
                table_hbm.at[idx_sl], rows_v.at[r, b], gsem[r]
            )

        def store_desc(gi, r, b):
            out_sl = out_hbm.at[pl.ds(base + (gi * K + b) * CHUNK, CHUNK)]
            return pltpu.make_async_copy(rows_v.at[r, b], out_sl, ssem[r])

        def fire_g(gi, r):
            for b in range(K):
                gather_desc(gi, r, b).start()

        def drain_g(gi, r):
            for b in range(K):
                gather_desc(gi, r, b).wait()

        def fire_s(gi, r):
            for b in range(K):
                store_desc(gi, r, b).start()

        def drain_s(gi, r):
            for b in range(K):
                store_desc(gi, r, b).wait()

        fire_g(0, 0)
        fire_g(1, 1)
        drain_g(0, 0)
        fire_s(0, 0)
        fire_g(2, 2)
        drain_g(1, 1)
        fire_s(1, 1)

        @pl.loop(2, G - 3, step=3)
        def _steady(i):
            for p in range(3):
                g = i + p
                r = (2 + p) % R
                rn = (r + 1) % R
                drain_s(g - 2, rn)
                fire_g(g + 1, rn)
                drain_g(g, r)
                fire_s(g, r)

        drain_s(45, 0)
        fire_g(48, 0)
        drain_g(47, 2)
        fire_s(47, 2)

        drain_s(46, 1)
        fire_g(49, 1)
        drain_g(48, 0)
        fire_s(48, 0)

        drain_s(47, 2)
        drain_g(49, 1)
        fire_s(49, 1)

        drain_s(48, 0)
        drain_s(49, 1)

    pl.run_scoped(
        body,
        pltpu.VMEM((B_PER_W,), jnp.int32),
        pltpu.VMEM((R, K, CHUNK, D), jnp.float32),
        *([pltpu.SemaphoreType.DMA] * 6),
    )


def kernel(x, table):
    out = _emb_lookup(x.reshape(-1), table)
    return out.reshape(x.shape + (D,))
